# baseline (speedup 1.0000x reference)
import functools

import jax
import jax.numpy as jnp
from jax import lax
from jax.experimental import pallas as pl
from jax.experimental.pallas import tpu as pltpu
from jax.experimental.pallas import tpu_sc as plsc

_VOCAB = 64
_EMBED = 256
_NC = 2
_NS = 16
_NW = _NC * _NS
_G = 16
_LAG = 16
_BLK = 2048
_N_TC = 344064


def _sc_embed(table, idx_flat):
    B = idx_flat.shape[0]
    b_per_w = B // _NW
    n_groups = b_per_w // _G
    mesh = plsc.VectorSubcoreMesh(core_axis_name="c", subcore_axis_name="s")

    @functools.partial(
        pl.kernel,
        mesh=mesh,
        out_type=jax.ShapeDtypeStruct((B, _EMBED), jnp.float32),
        scratch_types=[
            pltpu.VMEM((b_per_w,), jnp.int32),
            pltpu.VMEM((_VOCAB, _EMBED), jnp.float32),
            pltpu.SemaphoreType.DMA,
        ],
    )
    def k(table_hbm, idx_hbm, out_hbm, idx_v, table_v, sem):
        wid = lax.axis_index("s") * _NC + lax.axis_index("c")
        base = wid * b_per_w
        pltpu.sync_copy(table_hbm, table_v)
        pltpu.sync_copy(idx_hbm.at[pl.ds(base, b_per_w)], idx_v)

        def drain_one_group():
            pltpu.make_async_copy(
                table_v.at[pl.ds(0, _G)], out_hbm.at[pl.ds(base, _G)], sem
            ).wait()

        def body(g, carry):
            gvec = idx_v[pl.ds(g * _G, _G)]
            for l in range(_G):
                ridx = gvec[l]
                pltpu.async_copy(
                    table_v.at[pl.ds(ridx, 1)],
                    out_hbm.at[pl.ds(base + g * _G + l, 1)],
                    sem,
                )

            @pl.when(g >= _LAG)
            def _():
                drain_one_group()

            return carry

        lax.fori_loop(0, n_groups, body, 0)

        for _ in range(_LAG):
            drain_one_group()

    return k(table, idx_flat)


def _tc_embed(table, idx_flat):
    N = idx_flat.shape[0]

    def body(idx_ref, table_ref, out_ref):
        ids = idx_ref[...]
        onehot = (
            ids[:, None]
            == lax.broadcasted_iota(jnp.int32, (_BLK, _VOCAB), 1)
        ).astype(jnp.float32)
        out_ref[...] = jax.lax.dot(
            onehot, table_ref[...], precision=jax.lax.Precision.DEFAULT
        )

    return pl.pallas_call(
        body,
        grid=(N // _BLK,),
        in_specs=[
            pl.BlockSpec((_BLK,), lambda i: (i,)),
            pl.BlockSpec((_VOCAB, _EMBED), lambda i: (0, 0)),
        ],
        out_specs=pl.BlockSpec((_BLK, _EMBED), lambda i: (i, 0)),
        out_shape=jax.ShapeDtypeStruct((N, _EMBED), jnp.float32),
    )(idx_flat, table)


def kernel(smile_input, embed_weight):
    idx = smile_input.reshape(-1).astype(jnp.int32)
    n_sc = idx.shape[0] - _N_TC
    out_sc = _sc_embed(embed_weight, idx[:n_sc])
    out_tc = _tc_embed(embed_weight, idx[n_sc:])
    out = jnp.concatenate([out_sc, out_tc], axis=0)
    return out.reshape(smile_input.shape + (_EMBED,))


# speedup vs baseline: 2.7945x; 2.7945x over previous
import functools

import jax
import jax.numpy as jnp
from jax import lax
from jax.experimental import pallas as pl
from jax.experimental.pallas import tpu as pltpu
from jax.experimental.pallas import tpu_sc as plsc

_VOCAB = 64
_EMBED = 256
_NC = 2
_NS = 16
_NW = _NC * _NS
_G = 16
_LAG = 64


def _sc_embed(table, idx_flat):
    B = idx_flat.shape[0]
    b_per_w = B // _NW
    n_groups = b_per_w // _G
    mesh = plsc.VectorSubcoreMesh(core_axis_name="c", subcore_axis_name="s")

    @functools.partial(
        pl.kernel,
        mesh=mesh,
        out_type=jax.ShapeDtypeStruct((B, _EMBED), jnp.float32),
        scratch_types=[
            pltpu.VMEM((b_per_w,), jnp.int32),
            pltpu.VMEM((_VOCAB, _EMBED), jnp.float32),
            pltpu.SemaphoreType.DMA,
        ],
    )
    def k(table_hbm, idx_hbm, out_hbm, idx_v, table_v, sem):
        wid = lax.axis_index("s") * _NC + lax.axis_index("c")
        base = wid * b_per_w
        pltpu.sync_copy(table_hbm, table_v)
        pltpu.sync_copy(idx_hbm.at[pl.ds(base, b_per_w)], idx_v)

        def drain_one_group():
            pltpu.make_async_copy(
                table_v.at[pl.ds(0, _G)], out_hbm.at[pl.ds(base, _G)], sem
            ).wait()

        def body(g, carry):
            gvec = idx_v[pl.ds(g * _G, _G)]
            for l in range(_G):
                ridx = gvec[l]
                pltpu.async_copy(
                    table_v.at[pl.ds(ridx, 1)],
                    out_hbm.at[pl.ds(base + g * _G + l, 1)],
                    sem,
                )

            @pl.when(g >= _LAG)
            def _():
                drain_one_group()

            return carry

        lax.fori_loop(0, n_groups, body, 0)

        for _ in range(_LAG):
            drain_one_group()

    return k(table, idx_flat)


def kernel(smile_input, embed_weight):
    idx = smile_input.reshape(-1).astype(jnp.int32)
    out = _sc_embed(embed_weight, idx)
    return out.reshape(smile_input.shape + (_EMBED,))
